# baseline (speedup 1.0000x reference)
import functools

import numpy as np
import jax
import jax.numpy as jnp
from jax import lax
from jax.experimental import pallas as pl
from jax.experimental.pallas import tpu as pltpu
from jax.experimental.pallas import tpu_sc as plsc

_N = 16384
_D = 2048
_R = 745
_NW = 32
_DW = _D // _NW
_C = 512
_NCHUNK = _N // _C
_L = 16


def _build_combine_matrix() -> np.ndarray:
    m = np.zeros((_R, 197), np.float32)
    r = 0
    for s in range(60):
        m[r, 84 + s] = 1.0
        r += 1
    for mi in range(60):
        for y in range(3):
            m[r, 24 + mi] = 1.0
            m[r, 187 + y] = 1.0
            r += 1
    for d in range(31):
        for w in range(7):
            m[r, 144 + d] = 1.0
            m[r, 190 + w] = 1.0
            r += 1
    for h in range(24):
        for mo in range(12):
            m[r, 0 + h] = 1.0
            m[r, 175 + mo] = 1.0
            r += 1
    assert r == _R
    return m


_M_COMBINE = _build_combine_matrix()


def _combine_body(m_ref, w_ref, o_ref):
    o_ref[...] = jnp.dot(m_ref[...], w_ref[...],
                         preferred_element_type=jnp.float32)


def _combine_tables(m, wcat):
    return pl.pallas_call(
        _combine_body,
        out_shape=jax.ShapeDtypeStruct((_R, _D), jnp.float32),
    )(m, wcat)


def _sc_body(t_hbm, tf_hbm, out_hbm, table_v, tf_v, out_v0, out_v1,
             sem0, sem1):
    cid = lax.axis_index("c")
    sid = lax.axis_index("s")
    wid = sid * 2 + cid
    col0 = wid * _DW

    pltpu.sync_copy(t_hbm.at[:, pl.ds(col0, _DW)], table_v)

    lane = lax.iota(jnp.int32, _L)

    def compute_chunk(k, out_vb, sem_out):
        tok0 = k * _C
        pltpu.sync_copy(tf_hbm.at[:, pl.ds(tok0, _C)], tf_v)

        def group(g, _):
            base = g * _L
            hh = tf_v[0, pl.ds(base, _L)]
            mi = tf_v[1, pl.ds(base, _L)]
            se = tf_v[2, pl.ds(base, _L)]
            dy = tf_v[3, pl.ds(base, _L)]
            mo = tf_v[4, pl.ds(base, _L)]
            yr = tf_v[5, pl.ds(base, _L)]
            wd = tf_v[6, pl.ds(base, _L)]
            i0 = se
            i1 = mi * 3 + yr - 1949
            i2 = dy * 7 + wd + 233
            i3 = hh * 12 + mo + 456
            tok = lane + base
            for c in range(_DW):
                colv = jnp.full((_L,), c, dtype=jnp.int32)
                g0 = plsc.load_gather(table_v, [i0, colv])
                g1 = plsc.load_gather(table_v, [i1, colv])
                g2 = plsc.load_gather(table_v, [i2, colv])
                g3 = plsc.load_gather(table_v, [i3, colv])
                plsc.store_scatter(out_vb, [tok, colv], (g0 + g1) + (g2 + g3))
            return 0

        lax.fori_loop(0, _C // _L, group, 0)
        pltpu.async_copy(
            out_vb, out_hbm.at[pl.ds(tok0, _C), pl.ds(col0, _DW)], sem_out)

    def chunk_pair(p, _):
        k0 = p * 2

        @pl.when(p >= 1)
        def _():
            pltpu.make_async_copy(
                out_v0, out_hbm.at[pl.ds(0, _C), pl.ds(col0, _DW)],
                sem0).wait()

        compute_chunk(k0, out_v0, sem0)

        @pl.when(p >= 1)
        def _():
            pltpu.make_async_copy(
                out_v1, out_hbm.at[pl.ds(0, _C), pl.ds(col0, _DW)],
                sem1).wait()

        compute_chunk(k0 + 1, out_v1, sem1)
        return 0

    lax.fori_loop(0, _NCHUNK // 2, chunk_pair, 0)
    pltpu.make_async_copy(
        out_v0, out_hbm.at[pl.ds(0, _C), pl.ds(col0, _DW)], sem0).wait()
    pltpu.make_async_copy(
        out_v1, out_hbm.at[pl.ds(0, _C), pl.ds(col0, _DW)], sem1).wait()


def _sc_lookup(table, tf_t):
    mesh = plsc.VectorSubcoreMesh(core_axis_name="c", subcore_axis_name="s")
    run = functools.partial(
        pl.kernel,
        mesh=mesh,
        compiler_params=pltpu.CompilerParams(
            use_tc_tiling_on_sc=False, needs_layout_passes=False),
        out_type=jax.ShapeDtypeStruct((_N, _D), jnp.float32),
        scratch_types=[
            pltpu.VMEM((_R, _DW), jnp.float32),
            pltpu.VMEM((7, _C), jnp.int32),
            pltpu.VMEM((_C, _DW), jnp.float32),
            pltpu.VMEM((_C, _DW), jnp.float32),
            pltpu.SemaphoreType.DMA,
            pltpu.SemaphoreType.DMA,
        ],
    )(_sc_body)
    return run(table, tf_t)


def kernel(time_features, hour_w, minute_w, second_w, day_w, month_w,
           year_w, weekday_w):
    wcat = jnp.concatenate(
        [hour_w, minute_w, second_w, day_w, month_w, year_w, weekday_w],
        axis=0)
    table = _combine_tables(jnp.asarray(_M_COMBINE), wcat)
    tf_t = time_features.T
    return _sc_lookup(table, tf_t)


# speedup vs baseline: 8.4492x; 8.4492x over previous
import functools

import numpy as np
import jax
import jax.numpy as jnp
from jax import lax
from jax.experimental import pallas as pl
from jax.experimental.pallas import tpu as pltpu
from jax.experimental.pallas import tpu_sc as plsc

_N = 16384
_D = 2048
_R = 745
_NW = 32
_DW = _D // _NW
_C = 512
_NCHUNK = _N // _C
_L = 16


def _build_combine_matrix() -> np.ndarray:
    m = np.zeros((_R, 197), np.float32)
    r = 0
    for s in range(60):
        m[r, 84 + s] = 1.0
        r += 1
    for mi in range(60):
        for y in range(3):
            m[r, 24 + mi] = 1.0
            m[r, 187 + y] = 1.0
            r += 1
    for d in range(31):
        for w in range(7):
            m[r, 144 + d] = 1.0
            m[r, 190 + w] = 1.0
            r += 1
    for h in range(24):
        for mo in range(12):
            m[r, 0 + h] = 1.0
            m[r, 175 + mo] = 1.0
            r += 1
    assert r == _R
    return m


_M_COMBINE = _build_combine_matrix()


def _combine_body(m_ref, w_ref, o_ref):
    o_ref[...] = jnp.dot(m_ref[...], w_ref[...],
                         preferred_element_type=jnp.float32)


def _combine_tables(m, wcat):
    return pl.pallas_call(
        _combine_body,
        out_shape=jax.ShapeDtypeStruct((_R, _D), jnp.float32),
    )(m, wcat)


def _sc_body(t_hbm, tf_hbm, out_hbm, table_v, tf_v, out_v0, out_v1,
             sem0, sem1):
    cid = lax.axis_index("c")
    sid = lax.axis_index("s")
    wid = sid * 2 + cid
    col0 = wid * _DW

    pltpu.sync_copy(t_hbm.at[:, pl.ds(col0, _DW)], table_v)

    def compute_chunk(k, out_vb, sem_out):
        tok0 = k * _C
        pltpu.sync_copy(tf_hbm.at[:, pl.ds(tok0, _C)], tf_v)

        @plsc.parallel_loop(0, _C, step=_L)
        def _group(base):
            hh = tf_v[0, pl.ds(base, _L)]
            mi = tf_v[1, pl.ds(base, _L)]
            se = tf_v[2, pl.ds(base, _L)]
            dy = tf_v[3, pl.ds(base, _L)]
            mo = tf_v[4, pl.ds(base, _L)]
            yr = tf_v[5, pl.ds(base, _L)]
            wd = tf_v[6, pl.ds(base, _L)]
            i0v = se
            i1v = mi * 3 + yr - 1949
            i2v = dy * 7 + wd + 233
            i3v = hh * 12 + mo + 456
            for l in range(_L):
                a = i0v[l]
                b = i1v[l]
                c = i2v[l]
                d = i3v[l]
                for cg in range(_DW // _L):
                    sl = pl.ds(cg * _L, _L)
                    out_vb[base + l, sl] = (
                        (table_v[a, sl] + table_v[b, sl])
                        + (table_v[c, sl] + table_v[d, sl]))

        pltpu.async_copy(
            out_vb, out_hbm.at[pl.ds(tok0, _C), pl.ds(col0, _DW)], sem_out)

    def chunk_pair(p, _):
        k0 = p * 2

        @pl.when(p >= 1)
        def _():
            pltpu.make_async_copy(
                out_v0, out_hbm.at[pl.ds(0, _C), pl.ds(col0, _DW)],
                sem0).wait()

        compute_chunk(k0, out_v0, sem0)

        @pl.when(p >= 1)
        def _():
            pltpu.make_async_copy(
                out_v1, out_hbm.at[pl.ds(0, _C), pl.ds(col0, _DW)],
                sem1).wait()

        compute_chunk(k0 + 1, out_v1, sem1)
        return 0

    lax.fori_loop(0, _NCHUNK // 2, chunk_pair, 0)
    pltpu.make_async_copy(
        out_v0, out_hbm.at[pl.ds(0, _C), pl.ds(col0, _DW)], sem0).wait()
    pltpu.make_async_copy(
        out_v1, out_hbm.at[pl.ds(0, _C), pl.ds(col0, _DW)], sem1).wait()


def _sc_lookup(table, tf_t):
    mesh = plsc.VectorSubcoreMesh(core_axis_name="c", subcore_axis_name="s")
    run = functools.partial(
        pl.kernel,
        mesh=mesh,
        compiler_params=pltpu.CompilerParams(
            use_tc_tiling_on_sc=False, needs_layout_passes=False),
        out_type=jax.ShapeDtypeStruct((_N, _D), jnp.float32),
        scratch_types=[
            pltpu.VMEM((_R, _DW), jnp.float32),
            pltpu.VMEM((7, _C), jnp.int32),
            pltpu.VMEM((_C, _DW), jnp.float32),
            pltpu.VMEM((_C, _DW), jnp.float32),
            pltpu.SemaphoreType.DMA,
            pltpu.SemaphoreType.DMA,
        ],
    )(_sc_body)
    return run(table, tf_t)


def kernel(time_features, hour_w, minute_w, second_w, day_w, month_w,
           year_w, weekday_w):
    wcat = jnp.concatenate(
        [hour_w, minute_w, second_w, day_w, month_w, year_w, weekday_w],
        axis=0)
    table = _combine_tables(jnp.asarray(_M_COMBINE), wcat)
    tf_t = time_features.T
    return _sc_lookup(table, tf_t)
